# baseline (speedup 1.0000x reference)
import functools

import jax
import jax.numpy as jnp
from jax import lax
from jax.experimental import pallas as pl
from jax.experimental.pallas import tpu as pltpu
from jax.experimental.pallas import tpu_sc as plsc

N = 524288
EMB = 16
ND = 7
NCACHED = 4
NSTREAM = ND - NCACHED
NC = 2
NS = 16
NW = NC * NS
PER_W = N // NW
CHUNK = 128
NCH = PER_W // CHUNK
GROUPS = CHUNK // 16
OUTW = ND * EMB


def _sc_body(xf, t0, t1, t2, t3, t4, t5, t6, out,
             xbuf, idx_buf, outT_buf, rows_buf, tc0, tc1, tc2, tc3,
             sem_x, sem_g, sem_w):
    hbm_tables = (t4, t5, t6)
    caches = (tc0, tc1, tc2, tc3)
    wid = lax.axis_index("s") * NC + lax.axis_index("c")
    base = wid * PER_W
    lanes = lax.iota(jnp.int32, 16)
    zero16 = jnp.zeros((16,), jnp.int32)
    top = jnp.full((16,), 127, jnp.int32)

    pltpu.sync_copy(t0, tc0)
    pltpu.sync_copy(t1, tc1)
    pltpu.sync_copy(t2, tc2)
    pltpu.sync_copy(t3, tc3)

    def x_copy(c, b):
        return pltpu.make_async_copy(
            xf.at[:, pl.ds(base + c * CHUNK, CHUNK)], xbuf.at[b], sem_x)

    def stream_copies(b):
        return [pltpu.make_async_copy(
            hbm_tables[dd].at[idx_buf.at[b, dd]],
            rows_buf.at[b, dd], sem_g) for dd in range(NSTREAM)]

    def write_copy(c, b):
        ca = wid * NCH + c
        return pltpu.make_async_copy(
            outT_buf.at[b], out.at[:, ca], sem_w)

    x_copy(0, 0).start()

    def chunk_body(c, b):
        x_copy(c, b).wait()

        @pl.when(c + 1 < NCH)
        def _():
            x_copy(c + 1, 1 - b).start()

        @pl.when(c >= 2)
        def _():
            write_copy(c - 2, b).wait()

        xb = xbuf.at[b]
        ob = outT_buf.at[b]
        for j in range(GROUPS):
            o = j * 16
            xv = xb[0, pl.ds(o, 16)]
            yv = xb[1, pl.ds(o, 16)]
            zv = xb[2, pl.ds(o, 16)]
            ix = jnp.minimum(jnp.maximum((xv * 128.0).astype(jnp.int32), zero16), top)
            iy = jnp.minimum(jnp.maximum((yv * 128.0).astype(jnp.int32), zero16), top)
            iz = jnp.minimum(jnp.maximum((zv * 128.0).astype(jnp.int32), zero16), top)
            for d in range(ND):
                s = 6 - d
                bb = d + 1
                idx = ((ix >> s) << (2 * bb)) + ((iy >> s) << bb) + (iz >> s)
                if d < NCACHED:
                    src_base = idx * EMB
                    tcf = caches[d]
                    for e in range(EMB):
                        v = plsc.load_gather(tcf, [src_base + e])
                        r = d * EMB + e
                        ob[r // 8, r % 8, pl.ds(o, 16)] = v
                elif d < 6:
                    idx_buf[b, d - NCACHED, pl.ds(o, 16)] = idx
                else:
                    pltpu.async_copy(
                        t6.at[idx], rows_buf.at[b, 2, pl.ds(o, 16)], sem_g)

        for cp in stream_copies(b)[:2]:
            cp.start()

        @pl.when(c >= 1)
        def _():
            for cp in stream_copies(1 - b):
                cp.wait()
            obp = outT_buf.at[1 - b]
            for dd in range(NSTREAM):
                rf = rows_buf.at[1 - b, dd]
                for e in range(EMB):
                    ecol = jnp.full((16,), e, jnp.int32)
                    r = (NCACHED + dd) * EMB + e
                    for j in range(GROUPS):
                        v = plsc.load_gather(rf, [lanes + j * 16, ecol])
                        obp[r // 8, r % 8, pl.ds(j * 16, 16)] = v
            write_copy(c - 1, 1 - b).start()

        return 1 - b

    bl = lax.fori_loop(0, NCH, chunk_body, 0)

    last = NCH - 1
    lb = last % 2
    for cp in stream_copies(lb):
        cp.wait()
    obp = outT_buf.at[lb]
    for dd in range(NSTREAM):
        rf = rows_buf.at[lb, dd]
        for e in range(EMB):
            ecol = jnp.full((16,), e, jnp.int32)
            r = (NCACHED + dd) * EMB + e
            for j in range(GROUPS):
                v = plsc.load_gather(rf, [lanes + j * 16, ecol])
                obp[r // 8, r % 8, pl.ds(j * 16, 16)] = v
    write_copy(last, lb).start()
    write_copy(last - 1, 1 - lb).wait()
    write_copy(last, lb).wait()


@jax.jit
def kernel(x, table0, table1, table2, table3, table4, table5, table6):
    mesh = plsc.VectorSubcoreMesh(core_axis_name="c", subcore_axis_name="s")
    run = functools.partial(
        pl.kernel,
        mesh=mesh,
        out_type=jax.ShapeDtypeStruct((OUTW // 8, N // CHUNK, 8, CHUNK),
                                      jnp.float32),
        scratch_types=[
            pltpu.VMEM((2, 3, CHUNK), jnp.float32),
            pltpu.VMEM((2, NSTREAM, CHUNK), jnp.int32),
            pltpu.VMEM((2, OUTW // 8, 8, CHUNK), jnp.float32),
            pltpu.VMEM((2, NSTREAM, CHUNK, EMB), jnp.float32),
            pltpu.VMEM((8 * EMB,), jnp.float32),
            pltpu.VMEM((64 * EMB,), jnp.float32),
            pltpu.VMEM((512 * EMB,), jnp.float32),
            pltpu.VMEM((4096 * EMB,), jnp.float32),
            pltpu.SemaphoreType.DMA,
            pltpu.SemaphoreType.DMA,
            pltpu.SemaphoreType.DMA,
        ],
        compiler_params=pltpu.CompilerParams(
            use_tc_tiling_on_sc=False, needs_layout_passes=False),
    )(_sc_body)
    out4 = run(x.T, table0.reshape(-1), table1.reshape(-1),
               table2.reshape(-1), table3.reshape(-1),
               table4, table5, table6)
    return out4.transpose((1, 3, 0, 2)).reshape(N, OUTW)


# speedup vs baseline: 1.0013x; 1.0013x over previous
import functools

import jax
import jax.numpy as jnp
from jax import lax
from jax.experimental import pallas as pl
from jax.experimental.pallas import tpu as pltpu
from jax.experimental.pallas import tpu_sc as plsc

N = 524288
EMB = 16
ND = 7
NCACHED = 4
NSTREAM = ND - NCACHED
NC = 2
NS = 16
NW = NC * NS
PER_W = N // NW
CHUNK = 128
NCH = PER_W // CHUNK
GROUPS = CHUNK // 16
OUTW = ND * EMB


def _sc_body(xf, t0, t1, t2, t3, t4, t5, t6, out,
             xbuf, idx_buf, outT_buf, rows_buf, tc0, tc1, tc2, tc3,
             sem_x, sem_g, sem_w):
    hbm_tables = (t4, t5, t6)
    caches = (tc0, tc1, tc2, tc3)
    wid = lax.axis_index("s") * NC + lax.axis_index("c")
    base = wid * PER_W
    lanes = lax.iota(jnp.int32, 16)
    zero16 = jnp.zeros((16,), jnp.int32)
    top = jnp.full((16,), 127, jnp.int32)

    pltpu.sync_copy(t0, tc0)
    pltpu.sync_copy(t1, tc1)
    pltpu.sync_copy(t2, tc2)
    pltpu.sync_copy(t3, tc3)

    def x_copy(c, b):
        return pltpu.make_async_copy(
            xf.at[:, pl.ds(base + c * CHUNK, CHUNK)], xbuf.at[b], sem_x)

    def stream_copies(b):
        return [pltpu.make_async_copy(
            hbm_tables[dd].at[idx_buf.at[b, dd]],
            rows_buf.at[b, dd], sem_g) for dd in range(NSTREAM)]

    def write_copy(c, b):
        ca = wid * NCH + c
        return pltpu.make_async_copy(
            outT_buf.at[b], out.at[:, ca], sem_w)

    x_copy(0, 0).start()

    def chunk_body(c, b):
        x_copy(c, b).wait()

        @pl.when(c + 1 < NCH)
        def _():
            x_copy(c + 1, 1 - b).start()

        @pl.when(c >= 2)
        def _():
            write_copy(c - 2, b).wait()

        xb = xbuf.at[b]
        ob = outT_buf.at[b]
        for j in range(GROUPS):
            o = j * 16
            xv = xb[0, pl.ds(o, 16)]
            yv = xb[1, pl.ds(o, 16)]
            zv = xb[2, pl.ds(o, 16)]
            ix = jnp.minimum(jnp.maximum((xv * 128.0).astype(jnp.int32), zero16), top)
            iy = jnp.minimum(jnp.maximum((yv * 128.0).astype(jnp.int32), zero16), top)
            iz = jnp.minimum(jnp.maximum((zv * 128.0).astype(jnp.int32), zero16), top)
            for d in range(ND):
                s = 6 - d
                bb = d + 1
                idx = ((ix >> s) << (2 * bb)) + ((iy >> s) << bb) + (iz >> s)
                if d < NCACHED:
                    src_base = idx * EMB
                    tcf = caches[d]
                    for e in range(EMB):
                        v = plsc.load_gather(tcf, [src_base + e])
                        r = d * EMB + e
                        ob[r // 8, r % 8, pl.ds(o, 16)] = v
                else:
                    idx_buf[b, d - NCACHED, pl.ds(o, 16)] = idx

        for cp in stream_copies(b):
            cp.start()

        @pl.when(c >= 1)
        def _():
            for cp in stream_copies(1 - b):
                cp.wait()
            obp = outT_buf.at[1 - b]
            for dd in range(NSTREAM):
                rf = rows_buf.at[1 - b, dd]
                for e in range(EMB):
                    ecol = jnp.full((16,), e, jnp.int32)
                    r = (NCACHED + dd) * EMB + e
                    for j in range(GROUPS):
                        v = plsc.load_gather(rf, [lanes + j * 16, ecol])
                        obp[r // 8, r % 8, pl.ds(j * 16, 16)] = v
            write_copy(c - 1, 1 - b).start()

        return 1 - b

    bl = lax.fori_loop(0, NCH, chunk_body, 0)

    last = NCH - 1
    lb = last % 2
    for cp in stream_copies(lb):
        cp.wait()
    obp = outT_buf.at[lb]
    for dd in range(NSTREAM):
        rf = rows_buf.at[lb, dd]
        for e in range(EMB):
            ecol = jnp.full((16,), e, jnp.int32)
            r = (NCACHED + dd) * EMB + e
            for j in range(GROUPS):
                v = plsc.load_gather(rf, [lanes + j * 16, ecol])
                obp[r // 8, r % 8, pl.ds(j * 16, 16)] = v
    write_copy(last, lb).start()
    write_copy(last - 1, 1 - lb).wait()
    write_copy(last, lb).wait()


@jax.jit
def kernel(x, table0, table1, table2, table3, table4, table5, table6):
    mesh = plsc.VectorSubcoreMesh(core_axis_name="c", subcore_axis_name="s")
    run = functools.partial(
        pl.kernel,
        mesh=mesh,
        out_type=jax.ShapeDtypeStruct((OUTW // 8, N // CHUNK, 8, CHUNK),
                                      jnp.float32),
        scratch_types=[
            pltpu.VMEM((2, 3, CHUNK), jnp.float32),
            pltpu.VMEM((2, NSTREAM, CHUNK), jnp.int32),
            pltpu.VMEM((2, OUTW // 8, 8, CHUNK), jnp.float32),
            pltpu.VMEM((2, NSTREAM, CHUNK, EMB), jnp.float32),
            pltpu.VMEM((8 * EMB,), jnp.float32),
            pltpu.VMEM((64 * EMB,), jnp.float32),
            pltpu.VMEM((512 * EMB,), jnp.float32),
            pltpu.VMEM((4096 * EMB,), jnp.float32),
            pltpu.SemaphoreType.DMA,
            pltpu.SemaphoreType.DMA,
            pltpu.SemaphoreType.DMA,
        ],
        compiler_params=pltpu.CompilerParams(
            use_tc_tiling_on_sc=False, needs_layout_passes=False),
    )(_sc_body)
    out4 = run(x.T, table0.reshape(-1), table1.reshape(-1),
               table2.reshape(-1), table3.reshape(-1),
               table4, table5, table6)
    return out4.transpose((1, 3, 0, 2)).reshape(N, OUTW)
